# baseline (device time: 26771 ns/iter reference)
import jax
import jax.numpy as jnp
from jax import lax
from jax.experimental import pallas as pl
from jax.experimental.pallas import tpu as pltpu

N_DEV = 8
N_TOK = 2048
D_IN = 512
D_OUT = 1024
N_EXP = 32
E_LOCAL = N_EXP // N_DEV
CHUNK = N_TOK // N_DEV
CAP = 51
K = 64
QW = E_LOCAL * K


def kernel(x, router_W, route_idx, expert_W):

    def body(xbf, r_ref, wbf, out_ref,
             mcall_ref, crk8_ref, q_ref, yg_ref, sendc, recvc,
             send_sems, recv_sems):
        my_pos = lax.axis_index("i")

        with jax.named_scope("phase#p=barrier"):
            bar = pltpu.get_barrier_semaphore()
            for j in range(1, N_DEV):
                nbr = lax.rem(my_pos + j, N_DEV)
                pl.semaphore_signal(bar, inc=1, device_id=(nbr,),
                                    device_id_type=pl.DeviceIdType.MESH)
            pl.semaphore_wait(bar, N_DEV - 1)

        tri = (lax.broadcasted_iota(jnp.int32, (CHUNK, CHUNK), 0)
               >= lax.broadcasted_iota(jnp.int32, (CHUNK, CHUNK), 1)
               ).astype(jnp.bfloat16)
        grp = (lax.broadcasted_iota(jnp.int32, (N_EXP, N_DEV), 0) // E_LOCAL
               == lax.broadcasted_iota(jnp.int32, (N_EXP, N_DEV), 1)
               ).astype(jnp.bfloat16)
        r64 = lax.broadcasted_iota(jnp.int32, (CHUNK, K), 1)
        k_ids = r64
        EC = N_EXP + E_LOCAL
        ccol = lax.broadcasted_iota(jnp.int32, (CHUNK, EC), 1)
        e_cat = jnp.where(ccol < N_EXP, ccol,
                          my_pos * E_LOCAL + (ccol - N_EXP))

        base = jnp.zeros((1, EC), jnp.float32)
        for b in range(N_DEV):
          with jax.named_scope(f"phase#p=mask{b}"):
            rb = r_ref[pl.ds(b * CHUNK, CHUNK), :]
            oh = (rb == e_cat).astype(jnp.bfloat16)
            cnt = base + jnp.dot(tri, oh,
                                 preferred_element_type=jnp.float32)
            kept = oh * (cnt <= float(CAP)).astype(jnp.bfloat16)
            mcb = jnp.dot(kept[:, :N_EXP], grp,
                          preferred_element_type=jnp.float32
                          ).astype(jnp.bfloat16)
            mcall_ref[pl.ds(b * CHUNK, CHUNK), :] = mcb
            crk8_ref[pl.ds(b * CHUNK, CHUNK), :] = jnp.dot(
                tri, mcb, preferred_element_type=jnp.float32)
            cnt_my = cnt[:, N_EXP:]
            kept_my = kept[:, N_EXP:]
            for le in range(E_LOCAL):
                q_ref[pl.ds(b * CHUNK, CHUNK),
                      le * K:(le + 1) * K] = (
                    (cnt_my[:, le:le + 1].astype(jnp.int32) - 1 == r64)
                    .astype(jnp.bfloat16) * kept_my[:, le:le + 1])
            base = base + jnp.sum(oh.astype(jnp.float32), axis=0,
                                  keepdims=True)

        selrow = lax.broadcasted_iota(jnp.int32, (1, N_DEV), 1)

        def perm_t_at(chunk_start, col):
            selb = (selrow == col).astype(jnp.bfloat16)
            self_ = (selrow == col).astype(jnp.float32)
            mc = jnp.sum(mcall_ref[pl.ds(chunk_start, CHUNK), :] * selb,
                         axis=1, keepdims=True)
            rank = jnp.sum(crk8_ref[pl.ds(chunk_start, CHUNK), :] * self_,
                           axis=1, keepdims=True)
            eq = (rank.astype(jnp.int32) - 1 == k_ids)
            return eq.astype(jnp.bfloat16) * mc

        with jax.named_scope("phase#p=gemms"):
            xg4 = lax.dot_general(
                q_ref[...], xbf[...], (((0,), (0,)), ((), ())),
                preferred_element_type=jnp.float32
            ).astype(jnp.bfloat16)
            for le in range(E_LOCAL):
                yg_ref[le * K:(le + 1) * K, :] = jnp.dot(
                    xg4[le * K:(le + 1) * K, :], wbf[le],
                    preferred_element_type=jnp.float32).astype(jnp.bfloat16)

        sends = []
        for j in range(1, N_DEV):
          with jax.named_scope(f"phase#p=send{j}"):
            dst = lax.rem(my_pos + j, N_DEV)
            pt = perm_t_at(dst * CHUNK, my_pos)
            u4 = lax.dot_general(
                pt, q_ref[pl.ds(dst * CHUNK, CHUNK), :],
                (((0,), (0,)), ((), ())),
                preferred_element_type=jnp.float32
            ).astype(jnp.bfloat16)
            sendc[j - 1] = jnp.dot(
                u4, yg_ref[...],
                preferred_element_type=jnp.float32).astype(jnp.bfloat16)
            rdma = pltpu.make_async_remote_copy(
                src_ref=sendc.at[j - 1],
                dst_ref=recvc.at[j - 1],
                send_sem=send_sems.at[j - 1],
                recv_sem=recv_sems.at[j - 1],
                device_id=(dst,),
                device_id_type=pl.DeviceIdType.MESH,
            )
            rdma.start()
            sends.append(rdma)

        with jax.named_scope("phase#p=ownchunk"):
            acc = lax.dot_general(
                q_ref[pl.ds(my_pos * CHUNK, CHUNK), :], yg_ref[...],
                (((1,), (0,)), ((), ())),
                preferred_element_type=jnp.float32)

        for j in range(1, N_DEV):
          with jax.named_scope(f"phase#p=recv{j}"):
            src = lax.rem(my_pos + N_DEV - j, N_DEV)
            pr = perm_t_at(my_pos * CHUNK, src)
            recv = pltpu.make_async_remote_copy(
                src_ref=sendc.at[j - 1],
                dst_ref=recvc.at[j - 1],
                send_sem=send_sems.at[j - 1],
                recv_sem=recv_sems.at[j - 1],
                device_id=(src,),
                device_id_type=pl.DeviceIdType.MESH,
            )
            recv.wait_recv()
            acc = acc + jnp.dot(pr, recvc[j - 1],
                                preferred_element_type=jnp.float32)
        with jax.named_scope("phase#p=tail"):
            out_ref[...] = acc
            for rdma in sends:
                rdma.wait_send()

    return pl.pallas_call(
        body,
        out_shape=jax.ShapeDtypeStruct((CHUNK, D_OUT), jnp.float32),
        in_specs=[
            pl.BlockSpec(memory_space=pltpu.VMEM),
            pl.BlockSpec(memory_space=pltpu.VMEM),
            pl.BlockSpec(memory_space=pltpu.VMEM),
        ],
        out_specs=pl.BlockSpec(memory_space=pltpu.VMEM),
        scratch_shapes=[
            pltpu.VMEM((N_TOK, N_DEV), jnp.bfloat16),
            pltpu.VMEM((N_TOK, N_DEV), jnp.float32),
            pltpu.VMEM((N_TOK, QW), jnp.bfloat16),
            pltpu.VMEM((QW, D_OUT), jnp.bfloat16),
            pltpu.VMEM((N_DEV - 1, K, D_OUT), jnp.bfloat16),
            pltpu.VMEM((N_DEV - 1, K, D_OUT), jnp.bfloat16),
            pltpu.SemaphoreType.DMA((N_DEV - 1,)),
            pltpu.SemaphoreType.DMA((N_DEV - 1,)),
        ],
        compiler_params=pltpu.CompilerParams(collective_id=0),
    )(x.astype(jnp.bfloat16), route_idx, expert_W.astype(jnp.bfloat16))


# device time: 26078 ns/iter; 1.0266x vs baseline; 1.0266x over previous
import jax
import jax.numpy as jnp
from jax import lax
from jax.experimental import pallas as pl
from jax.experimental.pallas import tpu as pltpu

N_DEV = 8
N_TOK = 2048
D_IN = 512
D_OUT = 1024
N_EXP = 32
E_LOCAL = N_EXP // N_DEV
CHUNK = N_TOK // N_DEV
CAP = 51
K = 64
QW = E_LOCAL * K


def kernel(x, router_W, route_idx, expert_W):

    def body(x_ref, r_ref, w_ref, out_ref,
             xbf, wbf, mcall_ref, crk8_ref, q_ref, yg_ref, sendc, recvc,
             send_sems, recv_sems):
        my_pos = lax.axis_index("i")

        with jax.named_scope("phase#p=barrier"):
            bar = pltpu.get_barrier_semaphore()
            for j in range(1, N_DEV):
                nbr = lax.rem(my_pos + j, N_DEV)
                pl.semaphore_signal(bar, inc=1, device_id=(nbr,),
                                    device_id_type=pl.DeviceIdType.MESH)
            pl.semaphore_wait(bar, N_DEV - 1)

        with jax.named_scope("phase#p=casts"):
            xbf[...] = x_ref[...].astype(jnp.bfloat16)
            wbf[...] = w_ref[...].astype(jnp.bfloat16)

        tri = (lax.broadcasted_iota(jnp.int32, (CHUNK, CHUNK), 0)
               >= lax.broadcasted_iota(jnp.int32, (CHUNK, CHUNK), 1)
               ).astype(jnp.bfloat16)
        grp = (lax.broadcasted_iota(jnp.int32, (N_EXP, N_DEV), 0) // E_LOCAL
               == lax.broadcasted_iota(jnp.int32, (N_EXP, N_DEV), 1)
               ).astype(jnp.bfloat16)
        r64 = lax.broadcasted_iota(jnp.int32, (CHUNK, K), 1)
        k_ids = r64
        EC = N_EXP + E_LOCAL
        ccol = lax.broadcasted_iota(jnp.int32, (CHUNK, EC), 1)
        e_cat = jnp.where(ccol < N_EXP, ccol,
                          my_pos * E_LOCAL + (ccol - N_EXP))

        base = jnp.zeros((1, EC), jnp.float32)
        for b in range(N_DEV):
          with jax.named_scope(f"phase#p=mask{b}"):
            rb = r_ref[pl.ds(b * CHUNK, CHUNK), :]
            oh = (rb == e_cat).astype(jnp.bfloat16)
            cnt = base + jnp.dot(tri, oh,
                                 preferred_element_type=jnp.float32)
            kept = oh * (cnt <= float(CAP)).astype(jnp.bfloat16)
            mcb = jnp.dot(kept[:, :N_EXP], grp,
                          preferred_element_type=jnp.float32
                          ).astype(jnp.bfloat16)
            mcall_ref[pl.ds(b * CHUNK, CHUNK), :] = mcb
            crk8_ref[pl.ds(b * CHUNK, CHUNK), :] = jnp.dot(
                tri, mcb, preferred_element_type=jnp.float32)
            cnt_my = cnt[:, N_EXP:]
            kept_my = kept[:, N_EXP:]
            for le in range(E_LOCAL):
                q_ref[pl.ds(b * CHUNK, CHUNK),
                      le * K:(le + 1) * K] = (
                    (cnt_my[:, le:le + 1].astype(jnp.int32) - 1 == r64)
                    .astype(jnp.bfloat16) * kept_my[:, le:le + 1])
            base = base + jnp.sum(oh.astype(jnp.float32), axis=0,
                                  keepdims=True)

        selrow = lax.broadcasted_iota(jnp.int32, (1, N_DEV), 1)

        def perm_t_at(chunk_start, col):
            selb = (selrow == col).astype(jnp.bfloat16)
            self_ = (selrow == col).astype(jnp.float32)
            mc = jnp.sum(mcall_ref[pl.ds(chunk_start, CHUNK), :] * selb,
                         axis=1, keepdims=True)
            rank = jnp.sum(crk8_ref[pl.ds(chunk_start, CHUNK), :] * self_,
                           axis=1, keepdims=True)
            eq = (rank.astype(jnp.int32) - 1 == k_ids)
            return eq.astype(jnp.bfloat16) * mc

        with jax.named_scope("phase#p=gemms"):
            xg4 = lax.dot_general(
                q_ref[...], xbf[...], (((0,), (0,)), ((), ())),
                preferred_element_type=jnp.float32
            ).astype(jnp.bfloat16)
            for le in range(E_LOCAL):
                yg_ref[le * K:(le + 1) * K, :] = jnp.dot(
                    xg4[le * K:(le + 1) * K, :], wbf[le],
                    preferred_element_type=jnp.float32).astype(jnp.bfloat16)

        sends = []
        for j in range(1, N_DEV):
          with jax.named_scope(f"phase#p=send{j}"):
            dst = lax.rem(my_pos + j, N_DEV)
            pt = perm_t_at(dst * CHUNK, my_pos)
            u4 = lax.dot_general(
                pt, q_ref[pl.ds(dst * CHUNK, CHUNK), :],
                (((0,), (0,)), ((), ())),
                preferred_element_type=jnp.float32
            ).astype(jnp.bfloat16)
            sendc[j - 1] = jnp.dot(
                u4, yg_ref[...],
                preferred_element_type=jnp.float32).astype(jnp.bfloat16)
            rdma = pltpu.make_async_remote_copy(
                src_ref=sendc.at[j - 1],
                dst_ref=recvc.at[j - 1],
                send_sem=send_sems.at[j - 1],
                recv_sem=recv_sems.at[j - 1],
                device_id=(dst,),
                device_id_type=pl.DeviceIdType.MESH,
            )
            rdma.start()
            sends.append(rdma)

        with jax.named_scope("phase#p=ownchunk"):
            acc = lax.dot_general(
                q_ref[pl.ds(my_pos * CHUNK, CHUNK), :], yg_ref[...],
                (((1,), (0,)), ((), ())),
                preferred_element_type=jnp.float32)

        for j in range(1, N_DEV):
          with jax.named_scope(f"phase#p=recv{j}"):
            src = lax.rem(my_pos + N_DEV - j, N_DEV)
            pr = perm_t_at(my_pos * CHUNK, src)
            recv = pltpu.make_async_remote_copy(
                src_ref=sendc.at[j - 1],
                dst_ref=recvc.at[j - 1],
                send_sem=send_sems.at[j - 1],
                recv_sem=recv_sems.at[j - 1],
                device_id=(src,),
                device_id_type=pl.DeviceIdType.MESH,
            )
            recv.wait_recv()
            acc = acc + jnp.dot(pr, recvc[j - 1],
                                preferred_element_type=jnp.float32)
        with jax.named_scope("phase#p=tail"):
            out_ref[...] = acc
            for rdma in sends:
                rdma.wait_send()

    return pl.pallas_call(
        body,
        out_shape=jax.ShapeDtypeStruct((CHUNK, D_OUT), jnp.float32),
        in_specs=[
            pl.BlockSpec(memory_space=pltpu.VMEM),
            pl.BlockSpec(memory_space=pltpu.VMEM),
            pl.BlockSpec(memory_space=pltpu.VMEM),
        ],
        out_specs=pl.BlockSpec(memory_space=pltpu.VMEM),
        scratch_shapes=[
            pltpu.VMEM((N_TOK, D_IN), jnp.bfloat16),
            pltpu.VMEM((E_LOCAL, D_IN, D_OUT), jnp.bfloat16),
            pltpu.VMEM((N_TOK, N_DEV), jnp.bfloat16),
            pltpu.VMEM((N_TOK, N_DEV), jnp.float32),
            pltpu.VMEM((N_TOK, QW), jnp.bfloat16),
            pltpu.VMEM((QW, D_OUT), jnp.bfloat16),
            pltpu.VMEM((N_DEV - 1, K, D_OUT), jnp.bfloat16),
            pltpu.VMEM((N_DEV - 1, K, D_OUT), jnp.bfloat16),
            pltpu.SemaphoreType.DMA((N_DEV - 1,)),
            pltpu.SemaphoreType.DMA((N_DEV - 1,)),
        ],
        compiler_params=pltpu.CompilerParams(collective_id=0),
    )(x, route_idx, expert_W)
